# Initial kernel scaffold; baseline (speedup 1.0000x reference)
import functools

import numpy as np
import jax
import jax.numpy as jnp
from jax import lax
from jax.experimental import pallas as pl
from jax.experimental.pallas import tpu as pltpu
from jax.experimental.pallas import tpu_sc as plsc

B = 4096
NT = 26
VOCAB = 100000
DIM = 32
NI = NT + 1
TOTAL = B * NT
NW = 32
PER_W = TOTAL // NW
CHUNK = 128
NCH = PER_W // CHUNK

_LI, _LJ = np.tril_indices(NI, k=-1)
_POS = jnp.asarray(_LI * NI + _LJ, dtype=jnp.int32)

BB = 512
GRID = B // BB


def _sc_gather(tables_flat, idx2d):
    mesh = plsc.VectorSubcoreMesh(core_axis_name="c", subcore_axis_name="s")

    @functools.partial(
        pl.kernel,
        mesh=mesh,
        out_type=jax.ShapeDtypeStruct((TOTAL, DIM), jnp.float32),
        scratch_types=[
            pltpu.VMEM((NCH, CHUNK), jnp.int32),
            pltpu.VMEM((PER_W, DIM), jnp.float32),
            pltpu.SemaphoreType.DMA,
        ],
    )
    def run(table_hbm, idx_hbm, out_hbm, idx_v, rows_v, sem):
        wid = lax.axis_index("s") * 2 + lax.axis_index("c")
        pltpu.sync_copy(idx_hbm.at[pl.ds(wid * NCH, NCH)], idx_v)

        def fire(j, c):
            pltpu.async_copy(
                table_hbm.at[idx_v.at[j]],
                rows_v.at[pl.ds(j * CHUNK, CHUNK)],
                sem,
            )
            return c

        lax.fori_loop(0, NCH, fire, 0)
        pltpu.make_async_copy(table_hbm.at[pl.ds(0, PER_W)], rows_v, sem).wait()
        pltpu.sync_copy(rows_v, out_hbm.at[pl.ds(wid * PER_W, PER_W)])

    return run(tables_flat, idx2d)


def _tc_body(num_ref, emb_ref, wb0, bb0, wb1, bb1, wb2, bb2,
             w0x, w0f, bt0, wt1, bt1, wt2, bt2, wt3, bt3, wt4, bt4, out_ref):
    dot = lambda a, b: lax.dot_general(
        a, b, (((1,), (0,)), ((), ())), preferred_element_type=jnp.float32)
    x = num_ref[...]
    x = jnp.maximum(dot(x, wb0[...]) + bb0[...], 0.0)
    x = jnp.maximum(dot(x, wb1[...]) + bb1[...], 0.0)
    x = jnp.maximum(dot(x, wb2[...]) + bb2[...], 0.0)
    feats = jnp.concatenate([x, emb_ref[...]], axis=1)
    f3 = feats.reshape(BB, NI, DIM)
    xact = lax.dot_general(
        f3, f3, (((2,), (2,)), ((0,), (0,))),
        preferred_element_type=jnp.float32)
    xflat = xact.reshape(BB, NI * NI)
    z = jnp.maximum(dot(x, w0x[...]) + dot(xflat, w0f[...]) + bt0[...], 0.0)
    z = jnp.maximum(dot(z, wt1[...]) + bt1[...], 0.0)
    z = jnp.maximum(dot(z, wt2[...]) + bt2[...], 0.0)
    z = jnp.maximum(dot(z, wt3[...]) + bt3[...], 0.0)
    out_ref[...] = dot(z, wt4[...]) + bt4[...]


def _tc_forward(num, emb2, wb0, bb0, wb1, bb1, wb2, bb2,
                w0x, w0f, bt0, wt1, bt1, wt2, bt2, wt3, bt3, wt4, bt4):
    full = lambda a: pl.BlockSpec(a.shape, lambda i: (0,) * a.ndim)
    weights = (wb0, bb0, wb1, bb1, wb2, bb2, w0x, w0f, bt0,
               wt1, bt1, wt2, bt2, wt3, bt3, wt4, bt4)
    return pl.pallas_call(
        _tc_body,
        grid=(GRID,),
        in_specs=[
            pl.BlockSpec((BB, num.shape[1]), lambda i: (i, 0)),
            pl.BlockSpec((BB, emb2.shape[1]), lambda i: (i, 0)),
            *[full(w) for w in weights],
        ],
        out_specs=pl.BlockSpec((BB, 1), lambda i: (i, 0)),
        out_shape=jax.ShapeDtypeStruct((B, 1), jnp.float32),
    )(num, emb2, *weights)


def kernel(numerical_features, categorical_features, embedding_tables,
           Wb0, bb0, Wb1, bb1, Wb2, bb2,
           Wt0, bt0, Wt1, bt1, Wt2, bt2, Wt3, bt3, Wt4, bt4):
    offs = (jnp.arange(NT, dtype=jnp.int32) * VOCAB)[None, :]
    idx2d = (categorical_features + offs).reshape(TOTAL // CHUNK, CHUNK)
    tables_flat = embedding_tables.reshape(NT * VOCAB, DIM)
    emb = _sc_gather(tables_flat, idx2d)
    emb2 = emb.reshape(B, NT * DIM)

    w0x = Wt0[:DIM]
    w0f = jnp.zeros((NI * NI, Wt0.shape[1]), jnp.float32).at[_POS].set(Wt0[DIM:])
    r1 = lambda v: v.reshape(1, -1)
    return _tc_forward(
        numerical_features, emb2, Wb0, r1(bb0), Wb1, r1(bb1), Wb2, r1(bb2),
        w0x, w0f, r1(bt0), Wt1, r1(bt1), Wt2, r1(bt2), Wt3, r1(bt3),
        Wt4, r1(bt4))


# speedup vs baseline: 2.2264x; 2.2264x over previous
import functools

import numpy as np
import jax
import jax.numpy as jnp
from jax import lax
from jax.experimental import pallas as pl
from jax.experimental.pallas import tpu as pltpu
from jax.experimental.pallas import tpu_sc as plsc

B = 4096
NT = 26
VOCAB = 100000
DIM = 32
NI = NT + 1
TOTAL = B * NT
NW = 32
PER_W = TOTAL // NW
CHUNK = 128
NCH = PER_W // CHUNK

_LI, _LJ = np.tril_indices(NI, k=-1)
_POS = np.asarray(_LI * NI + _LJ, dtype=np.int32)

BB = 512
GRID = B // BB


def _sc_gather(tables_flat, idx_flat):
    mesh = plsc.VectorSubcoreMesh(core_axis_name="c", subcore_axis_name="s")

    @functools.partial(
        pl.kernel,
        mesh=mesh,
        compiler_params=pltpu.CompilerParams(use_tc_tiling_on_sc=False),
        out_type=jax.ShapeDtypeStruct((TOTAL, DIM), jnp.float32),
        scratch_types=[
            pltpu.VMEM((PER_W,), jnp.int32),
            pltpu.VMEM((PER_W, DIM), jnp.float32),
            pltpu.SemaphoreType.DMA,
        ],
    )
    def run(table_hbm, idx_hbm, out_hbm, idx_v, rows_v, sem):
        wid = lax.axis_index("s") * 2 + lax.axis_index("c")
        pltpu.sync_copy(idx_hbm.at[pl.ds(wid * PER_W, PER_W)], idx_v)

        def fire(j, c):
            pltpu.async_copy(
                table_hbm.at[idx_v.at[pl.ds(j * CHUNK, CHUNK)]],
                rows_v.at[pl.ds(j * CHUNK, CHUNK)],
                sem,
            )
            return c

        lax.fori_loop(0, NCH, fire, 0)
        pltpu.make_async_copy(table_hbm.at[pl.ds(0, PER_W)], rows_v, sem).wait()
        pltpu.sync_copy(rows_v, out_hbm.at[pl.ds(wid * PER_W, PER_W)])

    return run(tables_flat, idx_flat)


def _tc_body(num_ref, emb_ref, wb0, bb0, wb1, bb1, wb2, bb2,
             w0x, w0f, bt0, wt1, bt1, wt2, bt2, wt3, bt3, wt4, bt4, out_ref):
    dot = lambda a, b: lax.dot_general(
        a, b, (((1,), (0,)), ((), ())), preferred_element_type=jnp.float32)
    x = num_ref[...]
    x = jnp.maximum(dot(x, wb0[...]) + bb0[...], 0.0)
    x = jnp.maximum(dot(x, wb1[...]) + bb1[...], 0.0)
    x = jnp.maximum(dot(x, wb2[...]) + bb2[...], 0.0)
    feats = jnp.concatenate([x, emb_ref[...]], axis=1)
    f3 = feats.reshape(BB, NI, DIM)
    xact = lax.dot_general(
        f3, f3, (((2,), (2,)), ((0,), (0,))),
        preferred_element_type=jnp.float32)
    xflat = xact.reshape(BB, NI * NI)
    z = jnp.maximum(dot(x, w0x[...]) + dot(xflat, w0f[...]) + bt0[...], 0.0)
    z = jnp.maximum(dot(z, wt1[...]) + bt1[...], 0.0)
    z = jnp.maximum(dot(z, wt2[...]) + bt2[...], 0.0)
    z = jnp.maximum(dot(z, wt3[...]) + bt3[...], 0.0)
    out_ref[...] = dot(z, wt4[...]) + bt4[...]


def _tc_forward(num, emb2, wb0, bb0, wb1, bb1, wb2, bb2,
                w0x, w0f, bt0, wt1, bt1, wt2, bt2, wt3, bt3, wt4, bt4):
    full = lambda a: pl.BlockSpec(a.shape, lambda i: (0,) * a.ndim)
    weights = (wb0, bb0, wb1, bb1, wb2, bb2, w0x, w0f, bt0,
               wt1, bt1, wt2, bt2, wt3, bt3, wt4, bt4)
    return pl.pallas_call(
        _tc_body,
        grid=(GRID,),
        in_specs=[
            pl.BlockSpec((BB, num.shape[1]), lambda i: (i, 0)),
            pl.BlockSpec((BB, emb2.shape[1]), lambda i: (i, 0)),
            *[full(w) for w in weights],
        ],
        out_specs=pl.BlockSpec((BB, 1), lambda i: (i, 0)),
        out_shape=jax.ShapeDtypeStruct((B, 1), jnp.float32),
    )(num, emb2, *weights)


def kernel(numerical_features, categorical_features, embedding_tables,
           Wb0, bb0, Wb1, bb1, Wb2, bb2,
           Wt0, bt0, Wt1, bt1, Wt2, bt2, Wt3, bt3, Wt4, bt4):
    offs = (jnp.arange(NT, dtype=jnp.int32) * VOCAB)[None, :]
    idx_flat = (categorical_features + offs).reshape(TOTAL)
    tables_flat = embedding_tables.reshape(NT * VOCAB, DIM)
    emb = _sc_gather(tables_flat, idx_flat)
    emb2 = emb.reshape(B, NT * DIM)

    w0x = Wt0[:DIM]
    w0f = jnp.zeros((NI * NI, Wt0.shape[1]), jnp.float32).at[_POS].set(Wt0[DIM:])
    r1 = lambda v: v.reshape(1, -1)
    return _tc_forward(
        numerical_features, emb2, Wb0, r1(bb0), Wb1, r1(bb1), Wb2, r1(bb2),
        w0x, w0f, r1(bt0), Wt1, r1(bt1), Wt2, r1(bt2), Wt3, r1(bt3),
        Wt4, r1(bt4))
